# baseline (speedup 1.0000x reference)
import functools

import jax
import jax.numpy as jnp
from jax import lax
from jax.experimental import pallas as pl
from jax.experimental.pallas import tpu as pltpu
from jax.experimental.pallas import tpu_sc as plsc

K = 16
RA = 256
NCHUNK = 256
CHUNK = 16


def _tc_block(xa_ref, xs_ref, sim_ref, cid_ref):
    xa = xa_ref[0]
    xs = xs_ref[0]
    sn = xs.shape[1]

    na = jnp.sqrt(jnp.sum(xa * xa, axis=0))
    nt = jnp.sqrt(jnp.sum(xs * xs, axis=0))

    num = lax.dot_general(xa, xs, (((0,), (0,)), ((), ())),
                          preferred_element_type=jnp.float32)
    sim = num / (na[:, None] * nt[None, :])
    sim_ref[...] = sim
    key = sim * sim

    cm = key[:, 0:NCHUNK]
    for j in range(1, CHUNK):
        cm = jnp.maximum(cm, key[:, j * NCHUNK:(j + 1) * NCHUNK])

    colf = lax.broadcasted_iota(jnp.int32, cm.shape, 1).astype(jnp.float32)
    for k in range(K):
        m = jnp.max(cm, axis=1, keepdims=True)
        hit = cm == m
        cidf = jnp.min(jnp.where(hit, colf, float(NCHUNK)), axis=1)
        cid_ref[:, k] = cidf.astype(jnp.int32)
        cm = jnp.where(colf == cidf[:, None], -1.0, cm)


def _tc_stage(x_actuators, x_sensors):
    b, s, a = x_actuators.shape
    sn = x_sensors.shape[2]
    nblk = a // RA
    return pl.pallas_call(
        _tc_block,
        grid=(b, nblk),
        in_specs=[
            pl.BlockSpec((1, s, RA), lambda bi, i: (bi, 0, i)),
            pl.BlockSpec((1, s, sn), lambda bi, i: (bi, 0, 0)),
        ],
        out_specs=[
            pl.BlockSpec((RA, sn), lambda bi, i: (bi * nblk + i, 0)),
            pl.BlockSpec((RA, K), lambda bi, i: (bi * nblk + i, 0)),
        ],
        out_shape=[
            jax.ShapeDtypeStruct((b * a, sn), jnp.float32),
            jax.ShapeDtypeStruct((b * a, K), jnp.int32),
        ],
    )(x_actuators, x_sensors)


def _tc_stage_one(xa, xs):
    s, a = xa.shape
    sn = xs.shape[1]
    nblk = a // RA
    return pl.pallas_call(
        _tc_block,
        grid=(1, nblk),
        in_specs=[
            pl.BlockSpec((1, s, RA), lambda bi, i: (0, 0, i)),
            pl.BlockSpec((1, s, sn), lambda bi, i: (0, 0, 0)),
        ],
        out_specs=[
            pl.BlockSpec((RA, sn), lambda bi, i: (i, 0)),
            pl.BlockSpec((RA, K), lambda bi, i: (i, 0)),
        ],
        out_shape=[
            jax.ShapeDtypeStruct((a, sn), jnp.float32),
            jax.ShapeDtypeStruct((a, K), jnp.int32),
        ],
    )(xa[None], xs[None])


NC = 2
NS = 16


def _row_topk(row_v, cids):
    def leaf(j):
        cols = cids + NCHUNK * j
        vals = plsc.load_gather(row_v, [cols])
        keys = vals * vals
        return plsc.sort_key_val(keys, cols, descending=True)

    def merge(a, b):
        ak, ac = a
        bk, bc = b
        rbk = lax.rev(bk, (0,))
        rbc = lax.rev(bc, (0,))
        m = ak >= rbk
        mk = jnp.where(m, ak, rbk)
        mc = jnp.where(m, ac, rbc)
        return plsc.sort_key_val(mk, mc, descending=True)

    def subtree(lo):
        m1 = merge(leaf(lo), leaf(lo + 1))
        m2 = merge(leaf(lo + 2), leaf(lo + 3))
        return merge(m1, m2)

    fk, fc = merge(merge(subtree(0), subtree(4)),
                   merge(subtree(8), subtree(12)))
    fv = plsc.load_gather(row_v, [fc])
    return fc, fv


def _sc_body(sim_hbm, cid_hbm, idx_hbm, val_hbm,
             row_a0, row_a1, row_b0, row_b1, cids_all, oidx_all, oval_all,
             sem_a0, sem_a1, sem_b0, sem_b1, sem_c):
    nc = NC
    nw = nc * NS
    nrows = sim_hbm.shape[0]
    rows_per_w = nrows // nw
    wid = lax.axis_index("s") * nc + lax.axis_index("c")
    base = wid * rows_per_w
    last = base + rows_per_w - 1

    nk = rows_per_w * K
    pltpu.async_copy(cid_hbm.at[pl.ds(base * K, nk)], cids_all, sem_c)
    pltpu.async_copy(sim_hbm.at[base], row_a0, sem_a0)
    pltpu.async_copy(sim_hbm.at[base + 1], row_a1, sem_a1)
    pltpu.make_async_copy(cid_hbm.at[pl.ds(base * K, nk)], cids_all,
                          sem_c).wait()

    def process(r, row_v):
        cids = cids_all[pl.ds(r * K, K)]
        fc, fv = _row_topk(row_v, cids)
        oidx_all[pl.ds(r * K, K)] = fc
        oval_all[pl.ds(r * K, K)] = fv

    def do_pair(r, bufs, sems, nxt_bufs, nxt_sems, nxt0, nxt1):
        pltpu.async_copy(sim_hbm.at[nxt0], nxt_bufs[0], nxt_sems[0])
        pltpu.async_copy(sim_hbm.at[nxt1], nxt_bufs[1], nxt_sems[1])
        pltpu.make_async_copy(sim_hbm.at[r], bufs[0], sems[0]).wait()
        pltpu.make_async_copy(sim_hbm.at[r], bufs[1], sems[1]).wait()
        process(r - base, bufs[0])
        process(r - base + 1, bufs[1])

    def quad_body(q, carry):
        r = base + 4 * q
        do_pair(r, (row_a0, row_a1), (sem_a0, sem_a1),
                (row_b0, row_b1), (sem_b0, sem_b1),
                r + 2, r + 3)
        do_pair(r + 2, (row_b0, row_b1), (sem_b0, sem_b1),
                (row_a0, row_a1), (sem_a0, sem_a1),
                jnp.minimum(r + 4, last), jnp.minimum(r + 5, last))
        return carry

    lax.fori_loop(0, rows_per_w // 4, quad_body, 0)
    pltpu.make_async_copy(sim_hbm.at[last], row_a0, sem_a0).wait()
    pltpu.make_async_copy(sim_hbm.at[last], row_a1, sem_a1).wait()

    pltpu.sync_copy(oidx_all, idx_hbm.at[pl.ds(base * K, nk)])
    pltpu.sync_copy(oval_all, val_hbm.at[pl.ds(base * K, nk)])


def _sc_stage(sim, cids):
    nrows = sim.shape[0]
    sn = sim.shape[1]
    nw = NC * NS
    rows_per_w = nrows // nw
    mesh = plsc.VectorSubcoreMesh(core_axis_name="c", subcore_axis_name="s",
                                  num_cores=NC, num_subcores=NS)
    f = pl.kernel(
        _sc_body,
        out_type=[
            jax.ShapeDtypeStruct((nrows * K,), jnp.int32),
            jax.ShapeDtypeStruct((nrows * K,), jnp.float32),
        ],
        mesh=mesh,
        compiler_params=pltpu.CompilerParams(needs_layout_passes=False),
        scratch_types=[
            pltpu.VMEM((sn,), jnp.float32),
            pltpu.VMEM((sn,), jnp.float32),
            pltpu.VMEM((sn,), jnp.float32),
            pltpu.VMEM((sn,), jnp.float32),
            pltpu.VMEM((rows_per_w * K,), jnp.int32),
            pltpu.VMEM((rows_per_w * K,), jnp.int32),
            pltpu.VMEM((rows_per_w * K,), jnp.float32),
            pltpu.SemaphoreType.DMA,
            pltpu.SemaphoreType.DMA,
            pltpu.SemaphoreType.DMA,
            pltpu.SemaphoreType.DMA,
            pltpu.SemaphoreType.DMA,
        ],
    )
    idx, val = f(sim, cids.reshape(-1))
    return idx.reshape(nrows, K), val.reshape(nrows, K)


def kernel(x_actuators, x_sensors):
    b, s, a = x_actuators.shape
    ah = a // 2
    idx_parts, val_parts = [], []
    for bi in range(b):
        parts = []
        for h in range(2):
            sim, cids = _tc_stage_one(
                x_actuators[bi, :, h * ah:(h + 1) * ah], x_sensors[bi])
            parts.append(_sc_stage(sim, cids))
        idx_parts.append(jnp.concatenate([p[0] for p in parts], axis=0))
        val_parts.append(jnp.concatenate([p[1] for p in parts], axis=0))
    indices = jnp.stack(idx_parts)
    values = jnp.stack(val_parts)
    target_nodes = indices.reshape(b, a * K)
    weights = values.reshape(b, a * K)
    source_nodes = jnp.tile(jnp.repeat(jnp.arange(a, dtype=jnp.int32), K)[None, :], (b, 1))
    edges = jnp.stack([source_nodes, target_nodes], axis=1)
    return edges, weights


# speedup vs baseline: 1.0278x; 1.0278x over previous
import functools

import jax
import jax.numpy as jnp
from jax import lax
from jax.experimental import pallas as pl
from jax.experimental.pallas import tpu as pltpu
from jax.experimental.pallas import tpu_sc as plsc

K = 16
RA = 256
NCHUNK = 256
CHUNK = 16


def _tc_block(xa_ref, xs_ref, sim_ref, cid_ref):
    xa = xa_ref[0]
    xs = xs_ref[0]
    sn = xs.shape[1]

    na = jnp.sqrt(jnp.sum(xa * xa, axis=0))
    nt = jnp.sqrt(jnp.sum(xs * xs, axis=0))

    num = lax.dot_general(xa, xs, (((0,), (0,)), ((), ())),
                          preferred_element_type=jnp.float32)
    sim = num / (na[:, None] * nt[None, :])
    sim_ref[...] = sim
    key = sim * sim

    cm = key[:, 0:NCHUNK]
    for j in range(1, CHUNK):
        cm = jnp.maximum(cm, key[:, j * NCHUNK:(j + 1) * NCHUNK])

    colf = lax.broadcasted_iota(jnp.int32, cm.shape, 1).astype(jnp.float32)
    for k in range(K):
        m = jnp.max(cm, axis=1, keepdims=True)
        hit = cm == m
        cidf = jnp.min(jnp.where(hit, colf, float(NCHUNK)), axis=1)
        cid_ref[:, k] = cidf.astype(jnp.int32)
        cm = jnp.where(colf == cidf[:, None], -1.0, cm)


def _tc_stage(x_actuators, x_sensors):
    b, s, a = x_actuators.shape
    sn = x_sensors.shape[2]
    nblk = a // RA
    return pl.pallas_call(
        _tc_block,
        grid=(b, nblk),
        in_specs=[
            pl.BlockSpec((1, s, RA), lambda bi, i: (bi, 0, i)),
            pl.BlockSpec((1, s, sn), lambda bi, i: (bi, 0, 0)),
        ],
        out_specs=[
            pl.BlockSpec((RA, sn), lambda bi, i: (bi * nblk + i, 0)),
            pl.BlockSpec((RA, K), lambda bi, i: (bi * nblk + i, 0)),
        ],
        out_shape=[
            jax.ShapeDtypeStruct((b * a, sn), jnp.float32),
            jax.ShapeDtypeStruct((b * a, K), jnp.int32),
        ],
    )(x_actuators, x_sensors)


def _tc_stage_one(xa, xs):
    s, a = xa.shape
    sn = xs.shape[1]
    nblk = a // RA
    return pl.pallas_call(
        _tc_block,
        grid=(1, nblk),
        in_specs=[
            pl.BlockSpec((1, s, RA), lambda bi, i: (0, 0, i)),
            pl.BlockSpec((1, s, sn), lambda bi, i: (0, 0, 0)),
        ],
        out_specs=[
            pl.BlockSpec((RA, sn), lambda bi, i: (i, 0)),
            pl.BlockSpec((RA, K), lambda bi, i: (i, 0)),
        ],
        out_shape=[
            jax.ShapeDtypeStruct((a, sn), jnp.float32),
            jax.ShapeDtypeStruct((a, K), jnp.int32),
        ],
    )(xa[None], xs[None])


NC = 2
NS = 16


def _row_topk(row_v, cids):
    def leaf(j):
        cols = cids + NCHUNK * j
        vals = plsc.load_gather(row_v, [cols])
        keys = vals * vals
        return plsc.sort_key_val(keys, cols, descending=True)

    def merge(a, b):
        ak, ac = a
        bk, bc = b
        rbk = lax.rev(bk, (0,))
        rbc = lax.rev(bc, (0,))
        m = ak >= rbk
        mk = jnp.where(m, ak, rbk)
        mc = jnp.where(m, ac, rbc)
        return plsc.sort_key_val(mk, mc, descending=True)

    def subtree(lo):
        m1 = merge(leaf(lo), leaf(lo + 1))
        m2 = merge(leaf(lo + 2), leaf(lo + 3))
        return merge(m1, m2)

    fk, fc = merge(merge(subtree(0), subtree(4)),
                   merge(subtree(8), subtree(12)))
    fv = plsc.load_gather(row_v, [fc])
    return fc, fv


def _sc_body(sim_hbm, cid_hbm, idx_hbm, val_hbm,
             row_a0, row_a1, row_b0, row_b1, cids_all, oidx_all, oval_all,
             sem_a0, sem_a1, sem_b0, sem_b1, sem_c):
    nc = NC
    nw = nc * NS
    nrows = sim_hbm.shape[0]
    rows_per_w = nrows // nw
    wid = lax.axis_index("s") * nc + lax.axis_index("c")
    base = wid * rows_per_w
    last = base + rows_per_w - 1

    nk = rows_per_w * K
    pltpu.async_copy(cid_hbm.at[pl.ds(base * K, nk)], cids_all, sem_c)
    pltpu.async_copy(sim_hbm.at[base], row_a0, sem_a0)
    pltpu.async_copy(sim_hbm.at[base + 1], row_a1, sem_a1)
    pltpu.make_async_copy(cid_hbm.at[pl.ds(base * K, nk)], cids_all,
                          sem_c).wait()

    def process(r, row_v):
        cids = cids_all[pl.ds(r * K, K)]
        fc, fv = _row_topk(row_v, cids)
        oidx_all[pl.ds(r * K, K)] = fc
        oval_all[pl.ds(r * K, K)] = fv

    def do_pair(r, bufs, sems, nxt_bufs, nxt_sems, nxt0, nxt1):
        pltpu.async_copy(sim_hbm.at[nxt0], nxt_bufs[0], nxt_sems[0])
        pltpu.async_copy(sim_hbm.at[nxt1], nxt_bufs[1], nxt_sems[1])
        pltpu.make_async_copy(sim_hbm.at[r], bufs[0], sems[0]).wait()
        pltpu.make_async_copy(sim_hbm.at[r], bufs[1], sems[1]).wait()
        process(r - base, bufs[0])
        process(r - base + 1, bufs[1])

    def quad_body(q, carry):
        r = base + 4 * q
        do_pair(r, (row_a0, row_a1), (sem_a0, sem_a1),
                (row_b0, row_b1), (sem_b0, sem_b1),
                r + 2, r + 3)
        do_pair(r + 2, (row_b0, row_b1), (sem_b0, sem_b1),
                (row_a0, row_a1), (sem_a0, sem_a1),
                jnp.minimum(r + 4, last), jnp.minimum(r + 5, last))
        return carry

    lax.fori_loop(0, rows_per_w // 4, quad_body, 0)
    pltpu.make_async_copy(sim_hbm.at[last], row_a0, sem_a0).wait()
    pltpu.make_async_copy(sim_hbm.at[last], row_a1, sem_a1).wait()

    pltpu.sync_copy(oidx_all, idx_hbm.at[pl.ds(base * K, nk)])
    pltpu.sync_copy(oval_all, val_hbm.at[pl.ds(base * K, nk)])


def _sc_stage(sim, cids):
    nrows = sim.shape[0]
    sn = sim.shape[1]
    nw = NC * NS
    rows_per_w = nrows // nw
    mesh = plsc.VectorSubcoreMesh(core_axis_name="c", subcore_axis_name="s",
                                  num_cores=NC, num_subcores=NS)
    f = pl.kernel(
        _sc_body,
        out_type=[
            jax.ShapeDtypeStruct((nrows * K,), jnp.int32),
            jax.ShapeDtypeStruct((nrows * K,), jnp.float32),
        ],
        mesh=mesh,
        compiler_params=pltpu.CompilerParams(needs_layout_passes=False),
        scratch_types=[
            pltpu.VMEM((sn,), jnp.float32),
            pltpu.VMEM((sn,), jnp.float32),
            pltpu.VMEM((sn,), jnp.float32),
            pltpu.VMEM((sn,), jnp.float32),
            pltpu.VMEM((rows_per_w * K,), jnp.int32),
            pltpu.VMEM((rows_per_w * K,), jnp.int32),
            pltpu.VMEM((rows_per_w * K,), jnp.float32),
            pltpu.SemaphoreType.DMA,
            pltpu.SemaphoreType.DMA,
            pltpu.SemaphoreType.DMA,
            pltpu.SemaphoreType.DMA,
            pltpu.SemaphoreType.DMA,
        ],
    )
    idx, val = f(sim, cids.reshape(-1))
    return idx.reshape(nrows, K), val.reshape(nrows, K)


def kernel(x_actuators, x_sensors):
    b, s, a = x_actuators.shape
    idx_parts, val_parts = [], []
    for bi in range(b):
        sim, cids = _tc_stage_one(x_actuators[bi], x_sensors[bi])
        idx, val = _sc_stage(sim, cids)
        idx_parts.append(idx)
        val_parts.append(val)
    indices = jnp.stack(idx_parts)
    values = jnp.stack(val_parts)
    target_nodes = indices.reshape(b, a * K)
    weights = values.reshape(b, a * K)
    source_nodes = jnp.tile(jnp.repeat(jnp.arange(a, dtype=jnp.int32), K)[None, :], (b, 1))
    edges = jnp.stack([source_nodes, target_nodes], axis=1)
    return edges, weights


# speedup vs baseline: 1.0293x; 1.0015x over previous
import jax
import jax.numpy as jnp
from jax import lax
from jax.experimental import pallas as pl
from jax.experimental.pallas import tpu as pltpu
from jax.experimental.pallas import tpu_sc as plsc

K = 16
RA = 256
NCHUNK = 256
CHUNK = 16


def _tc_block(xa_ref, xs_ref, sim_ref, cid_ref):
    xa = xa_ref[0]
    xs = xs_ref[0]
    sn = xs.shape[1]

    na = jnp.sqrt(jnp.sum(xa * xa, axis=0))
    nt = jnp.sqrt(jnp.sum(xs * xs, axis=0))

    num = lax.dot_general(xa, xs, (((0,), (0,)), ((), ())),
                          preferred_element_type=jnp.float32)
    sim = num / (na[:, None] * nt[None, :])
    sim_ref[...] = sim
    key = sim * sim

    cm = key[:, 0:NCHUNK]
    for j in range(1, CHUNK):
        cm = jnp.maximum(cm, key[:, j * NCHUNK:(j + 1) * NCHUNK])

    colf = lax.broadcasted_iota(jnp.int32, cm.shape, 1).astype(jnp.float32)
    for k in range(K):
        m = jnp.max(cm, axis=1, keepdims=True)
        hit = cm == m
        cidf = jnp.min(jnp.where(hit, colf, float(NCHUNK)), axis=1)
        cid_ref[:, k] = cidf.astype(jnp.int32)
        cm = jnp.where(colf == cidf[:, None], -1.0, cm)


def _tc_stage_one(xa, xs):
    s, a = xa.shape
    sn = xs.shape[1]
    nblk = a // RA
    return pl.pallas_call(
        _tc_block,
        grid=(1, nblk),
        in_specs=[
            pl.BlockSpec((1, s, RA), lambda bi, i: (0, 0, i)),
            pl.BlockSpec((1, s, sn), lambda bi, i: (0, 0, 0)),
        ],
        out_specs=[
            pl.BlockSpec((RA, sn), lambda bi, i: (i, 0)),
            pl.BlockSpec((RA, K), lambda bi, i: (i, 0)),
        ],
        out_shape=[
            jax.ShapeDtypeStruct((a, sn), jnp.float32),
            jax.ShapeDtypeStruct((a, K), jnp.int32),
        ],
    )(xa[None], xs[None])


NC = 2
NS = 16


def _row_topk(row_v, cids):
    def leaf(j):
        cols = cids + NCHUNK * j
        vals = plsc.load_gather(row_v, [cols])
        keys = vals * vals
        return plsc.sort_key_val(keys, cols, descending=True)

    def merge(a, b):
        ak, ac = a
        bk, bc = b
        rbk = lax.rev(bk, (0,))
        rbc = lax.rev(bc, (0,))
        m = ak >= rbk
        mk = jnp.where(m, ak, rbk)
        mc = jnp.where(m, ac, rbc)
        return plsc.sort_key_val(mk, mc, descending=True)

    def subtree(lo):
        m1 = merge(leaf(lo), leaf(lo + 1))
        m2 = merge(leaf(lo + 2), leaf(lo + 3))
        return merge(m1, m2)

    fk, fc = merge(merge(subtree(0), subtree(4)),
                   merge(subtree(8), subtree(12)))
    fv = plsc.load_gather(row_v, [fc])
    return fc, fv


def _sc_body(sim_hbm, cid_hbm, idx_hbm, val_hbm,
             row_a0, row_a1, row_b0, row_b1, cids_all, oidx_all, oval_all,
             sem_a0, sem_a1, sem_b0, sem_b1, sem_c):
    nc = NC
    nw = nc * NS
    nrows = sim_hbm.shape[0]
    rows_per_w = nrows // nw
    wid = lax.axis_index("s") * nc + lax.axis_index("c")
    base = wid * rows_per_w
    last = base + rows_per_w - 1

    nk = rows_per_w * K
    pltpu.async_copy(cid_hbm.at[pl.ds(base * K, nk)], cids_all, sem_c)
    pltpu.async_copy(sim_hbm.at[base], row_a0, sem_a0)
    pltpu.async_copy(sim_hbm.at[base + 1], row_a1, sem_a1)
    pltpu.make_async_copy(cid_hbm.at[pl.ds(base * K, nk)], cids_all,
                          sem_c).wait()

    def process(r, row_v):
        cids = cids_all[pl.ds(r * K, K)]
        fc, fv = _row_topk(row_v, cids)
        oidx_all[pl.ds(r * K, K)] = fc
        oval_all[pl.ds(r * K, K)] = fv

    def do_pair(r, bufs, sems, nxt_bufs, nxt_sems, nxt0, nxt1):
        pltpu.async_copy(sim_hbm.at[nxt0], nxt_bufs[0], nxt_sems[0])
        pltpu.async_copy(sim_hbm.at[nxt1], nxt_bufs[1], nxt_sems[1])
        pltpu.make_async_copy(sim_hbm.at[r], bufs[0], sems[0]).wait()
        pltpu.make_async_copy(sim_hbm.at[r + 1], bufs[1], sems[1]).wait()
        process(r - base, bufs[0])
        process(r - base + 1, bufs[1])

    def quad_body(q, carry):
        r = base + 4 * q
        do_pair(r, (row_a0, row_a1), (sem_a0, sem_a1),
                (row_b0, row_b1), (sem_b0, sem_b1),
                r + 2, r + 3)
        do_pair(r + 2, (row_b0, row_b1), (sem_b0, sem_b1),
                (row_a0, row_a1), (sem_a0, sem_a1),
                jnp.minimum(r + 4, last), jnp.minimum(r + 5, last))
        return carry

    lax.fori_loop(0, rows_per_w // 4, quad_body, 0)
    pltpu.make_async_copy(sim_hbm.at[last], row_a0, sem_a0).wait()
    pltpu.make_async_copy(sim_hbm.at[last], row_a1, sem_a1).wait()

    pltpu.sync_copy(oidx_all, idx_hbm.at[pl.ds(base * K, nk)])
    pltpu.sync_copy(oval_all, val_hbm.at[pl.ds(base * K, nk)])


def _sc_stage(sim, cids):
    nrows = sim.shape[0]
    sn = sim.shape[1]
    nw = NC * NS
    rows_per_w = nrows // nw
    mesh = plsc.VectorSubcoreMesh(core_axis_name="c", subcore_axis_name="s",
                                  num_cores=NC, num_subcores=NS)
    f = pl.kernel(
        _sc_body,
        out_type=[
            jax.ShapeDtypeStruct((nrows * K,), jnp.int32),
            jax.ShapeDtypeStruct((nrows * K,), jnp.float32),
        ],
        mesh=mesh,
        compiler_params=pltpu.CompilerParams(needs_layout_passes=False),
        scratch_types=[
            pltpu.VMEM((sn,), jnp.float32),
            pltpu.VMEM((sn,), jnp.float32),
            pltpu.VMEM((sn,), jnp.float32),
            pltpu.VMEM((sn,), jnp.float32),
            pltpu.VMEM((rows_per_w * K,), jnp.int32),
            pltpu.VMEM((rows_per_w * K,), jnp.int32),
            pltpu.VMEM((rows_per_w * K,), jnp.float32),
            pltpu.SemaphoreType.DMA,
            pltpu.SemaphoreType.DMA,
            pltpu.SemaphoreType.DMA,
            pltpu.SemaphoreType.DMA,
            pltpu.SemaphoreType.DMA,
        ],
    )
    idx, val = f(sim, cids.reshape(-1))
    return idx.reshape(nrows, K), val.reshape(nrows, K)


def kernel(x_actuators, x_sensors):
    b, s, a = x_actuators.shape
    idx_parts, val_parts = [], []
    for bi in range(b):
        sim, cids = _tc_stage_one(x_actuators[bi], x_sensors[bi])
        idx, val = _sc_stage(sim, cids)
        idx_parts.append(idx)
        val_parts.append(val)
    indices = jnp.stack(idx_parts)
    values = jnp.stack(val_parts)
    target_nodes = indices.reshape(b, a * K)
    weights = values.reshape(b, a * K)
    source_nodes = jnp.tile(jnp.repeat(jnp.arange(a, dtype=jnp.int32), K)[None, :], (b, 1))
    edges = jnp.stack([source_nodes, target_nodes], axis=1)
    return edges, weights
